# baseline (speedup 1.0000x reference)
import functools

import jax
import jax.numpy as jnp
from jax import lax
from jax.experimental import pallas as pl
from jax.experimental.pallas import tpu as pltpu
from jax.experimental.pallas import tpu_sc as plsc

NS = 16
XW = 8


def _gelu(x):
    return x * 0.5 * (1.0 + lax.erf(x * (2.0 ** -0.5)))


def _table_body(feat_ref, aw1_ref, table_ref):
    c = feat_ref.shape[1]
    f = feat_ref[0]
    ft = f.T
    a1a = aw1_ref[:, :c]
    fa = lax.dot_general(ft, a1a, (((1,), (1,)), ((), ())),
                         preferred_element_type=jnp.float32)
    table_ref[0] = jnp.concatenate([ft, fa], axis=1).astype(jnp.bfloat16)


def _consts_body(aw1_ref, pw2_ref, pb2_ref, ab1_ref, mt_ref, c0_ref):
    c = pw2_ref.shape[0]
    a1b = aw1_ref[:, c:]
    mt_ref[...] = lax.dot_general(pw2_ref[...], a1b, (((1,), (1,)), ((), ())),
                                  preferred_element_type=jnp.float32)
    c0_ref[...] = lax.dot_general(pb2_ref[...], a1b, (((1,), (1,)), ((), ())),
                                  preferred_element_type=jnp.float32) + ab1_ref[...]


def _select_body(xyz_ref, xyzt_ref, idx_ref, rel_ref, *, qa, n, q_base=0):
    b = pl.program_id(0)
    qi = pl.program_id(1)
    q0 = q_base + qi * qa
    xq = xyz_ref[0, pl.ds(q0, qa), :]
    d = []
    sq = None
    for k in range(3):
        dk = xq[:, k:k + 1] - xyzt_ref[0, k:k + 1, :]
        d.append(dk)
        sq = dk * dk if sq is None else sq + dk * dk
    dist = jnp.sqrt(sq)
    colid = lax.broadcasted_iota(jnp.int32, (qa, n), 1).astype(jnp.float32)
    dm = dist
    kth = None
    for _ in range(8):
        kth = jnp.min(dm, axis=1, keepdims=True)
        cand = jnp.where(dm == kth, colid, float(n))
        amin = jnp.min(cand, axis=1, keepdims=True)
        dm = jnp.where(colid == amin, jnp.inf, dm)
    in_box = ((jnp.abs(d[0]) <= kth) & (jnp.abs(d[1]) <= kth)
              & (jnp.abs(d[2]) <= kth))
    z = jnp.where(jnp.logical_not(in_box) | (colid == 0.0), 1.0, 0.0)
    cums = z
    sh = 1
    while sh < n:
        shifted = jnp.concatenate(
            [jnp.zeros((qa, sh), jnp.float32), cums[:, :n - sh]], axis=1)
        cums = cums + shifted
        sh *= 2
    cz_ex = cums - z
    nzt = cums[:, n - 1:n]
    pos = jnp.where(z > 0.0, cz_ex, nzt + (colid - cz_ex))
    siota = lax.broadcasted_iota(jnp.int32, (qa, NS, n), 1).astype(jnp.float32)
    oh = (pos[:, None, :] == siota).astype(jnp.float32)
    x8 = jnp.concatenate(
        [xyz_ref[0],
         lax.broadcasted_iota(jnp.int32, (n, 1), 0).astype(jnp.float32),
         jnp.zeros((n, XW - 4), jnp.float32)], axis=1)
    r8 = jnp.dot(oh.reshape(qa * NS, n), x8,
                 preferred_element_type=jnp.float32).reshape(qa, NS, XW)
    idx_ref[0] = (r8[:, :, 3] + jnp.float32(b * n)).astype(jnp.int32)
    xqp = jnp.concatenate([xq, jnp.zeros((qa, XW - 3), jnp.float32)], axis=1)
    rel_ref[0] = r8 - xqp[:, None, :]


def _make_gather(rows_total, width, n_chunks, chunk):
    mesh = plsc.VectorSubcoreMesh(core_axis_name="c", subcore_axis_name="s")
    info = plsc.get_sparse_core_info()
    nc = info.num_cores

    @functools.partial(
        pl.kernel,
        mesh=mesh,
        out_type=jax.ShapeDtypeStruct((rows_total, width), jnp.int32),
        scratch_types=[
            pltpu.VMEM((n_chunks, chunk), jnp.int32),
            pltpu.VMEM((chunk, width), jnp.int32),
            pltpu.VMEM((chunk, width), jnp.int32),
            pltpu.SemaphoreType.DMA,
            pltpu.SemaphoreType.DMA,
            pltpu.SemaphoreType.DMA,
            pltpu.SemaphoreType.DMA,
        ],
    )
    def gather_k(table_hbm, idx_hbm, out_hbm, idx_v, rows_v0, rows_v1,
                 sem_g0, sem_g1, sem_s0, sem_s1):
        wid = lax.axis_index("s") * nc + lax.axis_index("c")
        pltpu.sync_copy(idx_hbm.at[wid], idx_v)
        base = wid * (n_chunks * chunk)

        def body(i, carry):
            c0 = 2 * i

            @pl.when(i > 0)
            def _():
                pltpu.make_async_copy(
                    rows_v0, out_hbm.at[pl.ds(base, chunk)], sem_s0).wait()
                pltpu.make_async_copy(
                    rows_v1, out_hbm.at[pl.ds(base, chunk)], sem_s1).wait()

            g0 = pltpu.async_copy(table_hbm.at[idx_v.at[c0]], rows_v0, sem_g0)
            g1 = pltpu.async_copy(table_hbm.at[idx_v.at[c0 + 1]], rows_v1,
                                  sem_g0)
            g0.wait()
            g1.wait()
            pltpu.async_copy(
                rows_v0, out_hbm.at[pl.ds(base + c0 * chunk, chunk)], sem_s0)
            pltpu.async_copy(
                rows_v1, out_hbm.at[pl.ds(base + (c0 + 1) * chunk, chunk)],
                sem_s1)
            return carry

        lax.fori_loop(0, n_chunks // 2, body, 0)
        pltpu.make_async_copy(
            rows_v0, out_hbm.at[pl.ds(base, chunk)], sem_s0).wait()
        pltpu.make_async_copy(
            rows_v1, out_hbm.at[pl.ds(base, chunk)], sem_s1).wait()

    return gather_k


def _attn_body(gath_ref, rel_ref, pw1_ref, pb1_ref, mt_ref, c0_ref, aw2_ref,
               ab2_ref, fw1_ref, fb1_ref, fw2_ref, fb2_ref, out_ref, *, qb, c):
    g = gath_ref[0].astype(jnp.float32)
    featg = g[:, :c]
    fag = g[:, c:2 * c]
    rel = rel_ref[0]
    h = _gelu(jnp.dot(rel, pw1_ref[...],
                      preferred_element_type=jnp.float32) + pb1_ref[...])
    z = (jnp.dot(h, mt_ref[...], preferred_element_type=jnp.float32)
         + fag + c0_ref[...])
    h1 = _gelu(z).reshape(qb, NS, c)
    logits = (jnp.sum(h1 * aw2_ref[...][None, :, :], axis=-1)
              + ab2_ref[0, 0])
    m = jnp.max(logits, axis=-1, keepdims=True)
    e = jnp.exp(logits - m)
    w = e / jnp.sum(e, axis=-1, keepdims=True)
    out = jnp.sum(featg.reshape(qb, NS, c) * w[:, :, None], axis=1)
    f1 = _gelu(lax.dot_general(out, fw1_ref[...], (((1,), (1,)), ((), ())),
                               preferred_element_type=jnp.float32)
               + fb1_ref[...])
    f2 = (lax.dot_general(f1, fw2_ref[...], (((1,), (1,)), ((), ())),
                          preferred_element_type=jnp.float32) + fb2_ref[...])
    out_ref[0] = (out + f2).T


def kernel(xyz, features, pos_w1, pos_b1, pos_w2, pos_b2, attn_w1, attn_b1,
           attn_w2, attn_b2, ffn_w1, ffn_b1, ffn_w2, ffn_b2):
    b, c, n = features.shape
    width = 2 * c
    qa = 128
    qb = 128
    workers = 32
    chunk = 64
    n_chunks = n * NS // (workers * chunk)

    mt, c0 = pl.pallas_call(
        _consts_body,
        out_shape=[
            jax.ShapeDtypeStruct((c, c), jnp.float32),
            jax.ShapeDtypeStruct((1, c), jnp.float32),
        ],
    )(attn_w1, pos_w2, pos_b2.reshape(1, c), attn_b1.reshape(1, c))

    pw1p = jnp.pad(pos_w1, ((0, XW - 3), (0, 0)))
    xyzt = jnp.swapaxes(xyz, 1, 2)

    pieces = 2
    n2 = n // pieces
    n_chunks2 = n2 * NS // (workers * chunk)
    wi = width // 2
    gather_fn = _make_gather(n2 * NS, wi, n_chunks2, chunk)

    tables = []
    sels = []
    for bi in range(b):
        feat_b = lax.slice_in_dim(features, bi, bi + 1, axis=0)
        xyz_b = lax.slice_in_dim(xyz, bi, bi + 1, axis=0)
        xyzt_b = lax.slice_in_dim(xyzt, bi, bi + 1, axis=0)

        tables.append(pl.pallas_call(
            _table_body,
            grid=(1,),
            in_specs=[
                pl.BlockSpec((1, c, n), lambda i: (i, 0, 0)),
                pl.BlockSpec((c, 2 * c), lambda i: (0, 0)),
            ],
            out_specs=pl.BlockSpec((1, n, width), lambda i: (i, 0, 0)),
            out_shape=jax.ShapeDtypeStruct((1, n, width), jnp.bfloat16),
        )(feat_b, attn_w1))

        for p in range(pieces):
            sels.append(pl.pallas_call(
                functools.partial(_select_body, qa=qa, n=n, q_base=p * n2),
                grid=(1, n2 // qa),
                in_specs=[
                    pl.BlockSpec((1, n, 3), lambda i, j: (i, 0, 0)),
                    pl.BlockSpec((1, 3, n), lambda i, j: (i, 0, 0)),
                ],
                out_specs=[
                    pl.BlockSpec((1, qa, NS), lambda i, j: (i, j, 0)),
                    pl.BlockSpec((1, qa, NS, XW), lambda i, j: (i, j, 0, 0)),
                ],
                out_shape=[
                    jax.ShapeDtypeStruct((1, n2, NS), jnp.int32),
                    jax.ShapeDtypeStruct((1, n2, NS, XW), jnp.float32),
                ],
            )(xyz_b, xyzt_b))

    outs = []
    for bi in range(b):
        row = []
        for p in range(pieces):
            idx, rel = sels[bi * pieces + p]
            table_w = lax.bitcast_convert_type(
                tables[bi].reshape(n, wi, 2), jnp.int32)
            gath = gather_fn(table_w,
                             idx.reshape(workers, n_chunks2, chunk))

            out_p = pl.pallas_call(
                functools.partial(_attn_body, qb=qb, c=c),
                grid=(1, n2 // qb),
                in_specs=[
                    pl.BlockSpec((1, qb * NS, width), lambda i, j: (i, j, 0)),
                    pl.BlockSpec((1, qb * NS, XW), lambda i, j: (i, j, 0)),
                    pl.BlockSpec((XW, c), lambda i, j: (0, 0)),
                    pl.BlockSpec((1, c), lambda i, j: (0, 0)),
                    pl.BlockSpec((c, c), lambda i, j: (0, 0)),
                    pl.BlockSpec((1, c), lambda i, j: (0, 0)),
                    pl.BlockSpec((1, c), lambda i, j: (0, 0)),
                    pl.BlockSpec((1, 1), lambda i, j: (0, 0)),
                    pl.BlockSpec((c, c), lambda i, j: (0, 0)),
                    pl.BlockSpec((1, c), lambda i, j: (0, 0)),
                    pl.BlockSpec((c, c), lambda i, j: (0, 0)),
                    pl.BlockSpec((1, c), lambda i, j: (0, 0)),
                ],
                out_specs=pl.BlockSpec((1, c, qb), lambda i, j: (i, 0, j)),
                out_shape=jax.ShapeDtypeStruct((1, c, n2), jnp.float32),
            )(lax.bitcast_convert_type(
                  gath.reshape(1, n2 * NS, wi), jnp.bfloat16
              ).reshape(1, n2 * NS, width), rel.reshape(1, n2 * NS, XW),
              pw1p, pos_b1.reshape(1, c),
              mt, c0, attn_w2, attn_b2.reshape(1, 1),
              ffn_w1, ffn_b1.reshape(1, c), ffn_w2, ffn_b2.reshape(1, c))
            row.append(out_p)
        outs.append(jnp.concatenate(row, axis=2))

    return jnp.concatenate(outs, axis=0)


# speedup vs baseline: 2.1768x; 2.1768x over previous
import functools

import jax
import jax.numpy as jnp
from jax import lax
from jax.experimental import pallas as pl
from jax.experimental.pallas import tpu as pltpu
from jax.experimental.pallas import tpu_sc as plsc

NS = 16
XW = 8


def _gelu(x):
    return x * 0.5 * (1.0 + lax.erf(x * (2.0 ** -0.5)))


def _bf16_bits(x):
    u = lax.bitcast_convert_type(x, jnp.int32)
    return lax.shift_right_logical(
        u + 0x7FFF + (lax.shift_right_logical(u, 16) & 1), 16)


def _table_body(feat_ref, aw1_ref, table_ref):
    c = feat_ref.shape[1]
    f = feat_ref[0]
    ft = f.T
    a1a = aw1_ref[:, :c]
    fa = lax.dot_general(ft, a1a, (((1,), (1,)), ((), ())),
                         preferred_element_type=jnp.float32)
    table_ref[0] = (lax.shift_left(_bf16_bits(fa), 16)
                    | (_bf16_bits(ft) & 0xFFFF))


def _consts_body(aw1_ref, pw2_ref, pb2_ref, ab1_ref, mt_ref, c0_ref):
    c = pw2_ref.shape[0]
    a1b = aw1_ref[:, c:]
    mt_ref[...] = lax.dot_general(pw2_ref[...], a1b, (((1,), (1,)), ((), ())),
                                  preferred_element_type=jnp.float32)
    c0_ref[...] = lax.dot_general(pb2_ref[...], a1b, (((1,), (1,)), ((), ())),
                                  preferred_element_type=jnp.float32) + ab1_ref[...]


def _select_body(xyz_ref, xyzt_ref, idx_ref, rel_ref, *, qa, n, q_base=0):
    b = pl.program_id(0)
    qi = pl.program_id(1)
    q0 = q_base + qi * qa
    xq = xyz_ref[0, pl.ds(q0, qa), :]
    d = []
    sq = None
    for k in range(3):
        dk = xq[:, k:k + 1] - xyzt_ref[0, k:k + 1, :]
        d.append(dk)
        sq = dk * dk if sq is None else sq + dk * dk
    dist = jnp.sqrt(sq)
    colid = lax.broadcasted_iota(jnp.int32, (qa, n), 1).astype(jnp.float32)
    dm = dist
    kth = None
    for _ in range(8):
        kth = jnp.min(dm, axis=1, keepdims=True)
        cand = jnp.where(dm == kth, colid, float(n))
        amin = jnp.min(cand, axis=1, keepdims=True)
        dm = jnp.where(colid == amin, jnp.inf, dm)
    in_box = ((jnp.abs(d[0]) <= kth) & (jnp.abs(d[1]) <= kth)
              & (jnp.abs(d[2]) <= kth))
    z = jnp.where(jnp.logical_not(in_box) | (colid == 0.0), 1.0, 0.0)
    cums = z
    sh = 1
    while sh < n:
        shifted = jnp.concatenate(
            [jnp.zeros((qa, sh), jnp.float32), cums[:, :n - sh]], axis=1)
        cums = cums + shifted
        sh *= 2
    cz_ex = cums - z
    nzt = cums[:, n - 1:n]
    pos = jnp.where(z > 0.0, cz_ex, nzt + (colid - cz_ex))
    siota = lax.broadcasted_iota(jnp.int32, (qa, NS, n), 1).astype(jnp.float32)
    oh = (pos[:, None, :] == siota).astype(jnp.float32)
    x8 = jnp.concatenate(
        [xyz_ref[0],
         lax.broadcasted_iota(jnp.int32, (n, 1), 0).astype(jnp.float32),
         jnp.zeros((n, XW - 4), jnp.float32)], axis=1)
    r8 = jnp.dot(oh.reshape(qa * NS, n), x8,
                 preferred_element_type=jnp.float32).reshape(qa, NS, XW)
    idx_ref[0] = (r8[:, :, 3] + jnp.float32(b * n)).astype(jnp.int32)
    xqp = jnp.concatenate([xq, jnp.zeros((qa, XW - 3), jnp.float32)], axis=1)
    rel_ref[0] = r8 - xqp[:, None, :]


def _make_gather(rows_total, width, n_chunks, chunk):
    mesh = plsc.VectorSubcoreMesh(core_axis_name="c", subcore_axis_name="s")
    info = plsc.get_sparse_core_info()
    nc = info.num_cores

    @functools.partial(
        pl.kernel,
        mesh=mesh,
        out_type=jax.ShapeDtypeStruct((rows_total, width), jnp.int32),
        scratch_types=[
            pltpu.VMEM((n_chunks, chunk), jnp.int32),
            pltpu.VMEM((chunk, width), jnp.int32),
            pltpu.VMEM((chunk, width), jnp.int32),
            pltpu.SemaphoreType.DMA,
            pltpu.SemaphoreType.DMA,
            pltpu.SemaphoreType.DMA,
            pltpu.SemaphoreType.DMA,
        ],
    )
    def gather_k(table_hbm, idx_hbm, out_hbm, idx_v, rows_v0, rows_v1,
                 sem_g0, sem_g1, sem_s0, sem_s1):
        wid = lax.axis_index("s") * nc + lax.axis_index("c")
        pltpu.sync_copy(idx_hbm.at[wid], idx_v)
        base = wid * (n_chunks * chunk)

        def body(i, carry):
            c0 = 2 * i

            @pl.when(i > 0)
            def _():
                pltpu.make_async_copy(
                    rows_v0, out_hbm.at[pl.ds(base, chunk)], sem_s0).wait()
                pltpu.make_async_copy(
                    rows_v1, out_hbm.at[pl.ds(base, chunk)], sem_s1).wait()

            g0 = pltpu.async_copy(table_hbm.at[idx_v.at[c0]], rows_v0, sem_g0)
            g1 = pltpu.async_copy(table_hbm.at[idx_v.at[c0 + 1]], rows_v1,
                                  sem_g0)
            g0.wait()
            g1.wait()
            pltpu.async_copy(
                rows_v0, out_hbm.at[pl.ds(base + c0 * chunk, chunk)], sem_s0)
            pltpu.async_copy(
                rows_v1, out_hbm.at[pl.ds(base + (c0 + 1) * chunk, chunk)],
                sem_s1)
            return carry

        lax.fori_loop(0, n_chunks // 2, body, 0)
        pltpu.make_async_copy(
            rows_v0, out_hbm.at[pl.ds(base, chunk)], sem_s0).wait()
        pltpu.make_async_copy(
            rows_v1, out_hbm.at[pl.ds(base, chunk)], sem_s1).wait()

    return gather_k


def _attn_body(gath_ref, rel_ref, pw1_ref, pb1_ref, mt_ref, c0_ref, aw2_ref,
               ab2_ref, fw1_ref, fb1_ref, fw2_ref, fb2_ref, out_ref, *, qb, c):
    w = gath_ref[0]
    featg = lax.bitcast_convert_type(lax.shift_left(w, 16), jnp.float32)
    fag = lax.bitcast_convert_type(w & jnp.int32(-65536), jnp.float32)
    rel = rel_ref[0]
    h = _gelu(jnp.dot(rel, pw1_ref[...],
                      preferred_element_type=jnp.float32) + pb1_ref[...])
    z = (jnp.dot(h, mt_ref[...], preferred_element_type=jnp.float32)
         + fag + c0_ref[...])
    h1 = _gelu(z).reshape(qb, NS, c)
    logits = (jnp.sum(h1 * aw2_ref[...][None, :, :], axis=-1)
              + ab2_ref[0, 0])
    m = jnp.max(logits, axis=-1, keepdims=True)
    e = jnp.exp(logits - m)
    w = e / jnp.sum(e, axis=-1, keepdims=True)
    out = jnp.sum(featg.reshape(qb, NS, c) * w[:, :, None], axis=1)
    f1 = _gelu(lax.dot_general(out, fw1_ref[...], (((1,), (1,)), ((), ())),
                               preferred_element_type=jnp.float32)
               + fb1_ref[...])
    f2 = (lax.dot_general(f1, fw2_ref[...], (((1,), (1,)), ((), ())),
                          preferred_element_type=jnp.float32) + fb2_ref[...])
    out_ref[0] = (out + f2).T


def kernel(xyz, features, pos_w1, pos_b1, pos_w2, pos_b2, attn_w1, attn_b1,
           attn_w2, attn_b2, ffn_w1, ffn_b1, ffn_w2, ffn_b2):
    b, c, n = features.shape
    width = 2 * c
    qa = 128
    qb = 128
    workers = 32
    chunk = 64
    n_chunks = n * NS // (workers * chunk)

    mt, c0 = pl.pallas_call(
        _consts_body,
        out_shape=[
            jax.ShapeDtypeStruct((c, c), jnp.float32),
            jax.ShapeDtypeStruct((1, c), jnp.float32),
        ],
    )(attn_w1, pos_w2, pos_b2.reshape(1, c), attn_b1.reshape(1, c))

    pw1p = jnp.pad(pos_w1, ((0, XW - 3), (0, 0)))
    xyzt = jnp.swapaxes(xyz, 1, 2)

    pieces = 2
    n2 = n // pieces
    n_chunks2 = n2 * NS // (workers * chunk)
    wi = width // 2
    gather_fn = _make_gather(n2 * NS, wi, n_chunks2, chunk)

    tables = []
    sels = []
    for bi in range(b):
        feat_b = lax.slice_in_dim(features, bi, bi + 1, axis=0)
        xyz_b = lax.slice_in_dim(xyz, bi, bi + 1, axis=0)
        xyzt_b = lax.slice_in_dim(xyzt, bi, bi + 1, axis=0)

        tables.append(pl.pallas_call(
            _table_body,
            grid=(1,),
            in_specs=[
                pl.BlockSpec((1, c, n), lambda i: (i, 0, 0)),
                pl.BlockSpec((c, 2 * c), lambda i: (0, 0)),
            ],
            out_specs=pl.BlockSpec((1, n, wi), lambda i: (i, 0, 0)),
            out_shape=jax.ShapeDtypeStruct((1, n, wi), jnp.int32),
        )(feat_b, attn_w1))

        for p in range(pieces):
            sels.append(pl.pallas_call(
                functools.partial(_select_body, qa=qa, n=n, q_base=p * n2),
                grid=(1, n2 // qa),
                in_specs=[
                    pl.BlockSpec((1, n, 3), lambda i, j: (i, 0, 0)),
                    pl.BlockSpec((1, 3, n), lambda i, j: (i, 0, 0)),
                ],
                out_specs=[
                    pl.BlockSpec((1, qa, NS), lambda i, j: (i, j, 0)),
                    pl.BlockSpec((1, qa, NS, XW), lambda i, j: (i, j, 0, 0)),
                ],
                out_shape=[
                    jax.ShapeDtypeStruct((1, n2, NS), jnp.int32),
                    jax.ShapeDtypeStruct((1, n2, NS, XW), jnp.float32),
                ],
            )(xyz_b, xyzt_b))

    outs = []
    for bi in range(b):
        row = []
        for p in range(pieces):
            idx, rel = sels[bi * pieces + p]
            gath = gather_fn(tables[bi].reshape(n, wi),
                             idx.reshape(workers, n_chunks2, chunk))

            out_p = pl.pallas_call(
                functools.partial(_attn_body, qb=qb, c=c),
                grid=(1, n2 // qb),
                in_specs=[
                    pl.BlockSpec((1, qb * NS, wi), lambda i, j: (i, j, 0)),
                    pl.BlockSpec((1, qb * NS, XW), lambda i, j: (i, j, 0)),
                    pl.BlockSpec((XW, c), lambda i, j: (0, 0)),
                    pl.BlockSpec((1, c), lambda i, j: (0, 0)),
                    pl.BlockSpec((c, c), lambda i, j: (0, 0)),
                    pl.BlockSpec((1, c), lambda i, j: (0, 0)),
                    pl.BlockSpec((1, c), lambda i, j: (0, 0)),
                    pl.BlockSpec((1, 1), lambda i, j: (0, 0)),
                    pl.BlockSpec((c, c), lambda i, j: (0, 0)),
                    pl.BlockSpec((1, c), lambda i, j: (0, 0)),
                    pl.BlockSpec((c, c), lambda i, j: (0, 0)),
                    pl.BlockSpec((1, c), lambda i, j: (0, 0)),
                ],
                out_specs=pl.BlockSpec((1, c, qb), lambda i, j: (i, 0, j)),
                out_shape=jax.ShapeDtypeStruct((1, c, n2), jnp.float32),
            )(gath.reshape(1, n2 * NS, wi), rel.reshape(1, n2 * NS, XW),
              pw1p, pos_b1.reshape(1, c),
              mt, c0, attn_w2, attn_b2.reshape(1, 1),
              ffn_w1, ffn_b1.reshape(1, c), ffn_w2, ffn_b2.reshape(1, c))
            row.append(out_p)
        outs.append(jnp.concatenate(row, axis=2))

    return jnp.concatenate(outs, axis=0)
